# baseline (speedup 1.0000x reference)
import functools

import jax
import jax.numpy as jnp
from jax import lax
from jax.experimental import pallas as pl
from jax.experimental.pallas import tpu as pltpu
from jax.experimental.pallas import tpu_sc as plsc

DIM = 64
NUM_CORES = 2
NUM_SUBCORES = 16
NUM_WORKERS = NUM_CORES * NUM_SUBCORES
NBUF = 4


def kernel(x, table):
    batch, seq = x.shape
    rows_per_w = batch // NUM_WORKERS
    n_groups = rows_per_w // NBUF
    assert n_groups * NBUF == rows_per_w
    mesh = plsc.VectorSubcoreMesh(core_axis_name="c", subcore_axis_name="s")

    @functools.partial(
        pl.kernel,
        mesh=mesh,
        out_type=jax.ShapeDtypeStruct((batch, seq, DIM), jnp.float32),
        scratch_types=[
            pltpu.VMEM((rows_per_w, seq), jnp.int32),
            pltpu.VMEM((NBUF, seq, DIM), jnp.float32),
            pltpu.SemaphoreType.DMA((NBUF,)),
            pltpu.SemaphoreType.DMA((NBUF,)),
        ],
        compiler_params=pltpu.CompilerParams(use_tc_tiling_on_sc=False),
    )
    def k(x_hbm, table_hbm, out_hbm, idx_v, rows_v, gsem, ssem):
        wid = lax.axis_index("s") * NUM_CORES + lax.axis_index("c")
        row0 = wid * rows_per_w
        pltpu.sync_copy(x_hbm.at[pl.ds(row0, rows_per_w)], idx_v)

        def gather_copy(i, b):
            r = i * NBUF + b
            return pltpu.make_async_copy(
                table_hbm.at[idx_v.at[r]],
                rows_v.at[b],
                gsem.at[b],
            )

        def store_copy(i, b):
            r = i * NBUF + b
            return pltpu.make_async_copy(
                rows_v.at[b],
                out_hbm.at[row0 + r],
                ssem.at[b],
            )

        for b in range(NBUF):
            gather_copy(0, b).start()

        def body(i, carry):
            for b in range(NBUF):
                gather_copy(i, b).wait()
                store_copy(i, b).start()
            for b in range(NBUF):
                store_copy(i, b).wait()
                gather_copy(i + 1, b).start()
            return carry

        lax.fori_loop(0, n_groups - 1, body, 0)

        last = n_groups - 1
        for b in range(NBUF):
            gather_copy(last, b).wait()
            store_copy(last, b).start()
        for b in range(NBUF):
            store_copy(last, b).wait()

    return k(x, table)


# speedup vs baseline: 1.3276x; 1.3276x over previous
import functools

import jax
import jax.numpy as jnp
from jax import lax
from jax.experimental import pallas as pl
from jax.experimental.pallas import tpu as pltpu
from jax.experimental.pallas import tpu_sc as plsc

DIM = 64
PITCH = 128
NUM_CORES = 2
NUM_SUBCORES = 16
NUM_WORKERS = NUM_CORES * NUM_SUBCORES
NBUF = 4


def kernel(x, table):
    batch, seq = x.shape
    rows_per_w = batch // NUM_WORKERS
    n_groups = rows_per_w // NBUF
    assert n_groups * NBUF == rows_per_w
    n_flat = batch * seq
    mesh = plsc.VectorSubcoreMesh(core_axis_name="c", subcore_axis_name="s")

    @functools.partial(
        pl.kernel,
        mesh=mesh,
        out_type=jax.ShapeDtypeStruct((n_flat, PITCH), jnp.float32),
        scratch_types=[
            pltpu.VMEM((rows_per_w, seq), jnp.int32),
            pltpu.VMEM((NBUF, seq, DIM), jnp.float32),
            pltpu.SemaphoreType.DMA((NBUF,)),
            pltpu.SemaphoreType.DMA((NBUF,)),
        ],
        compiler_params=pltpu.CompilerParams(use_tc_tiling_on_sc=False),
    )
    def k(x_hbm, table_hbm, out_hbm, idx_v, rows_v, gsem, ssem):
        wid = lax.axis_index("s") * NUM_CORES + lax.axis_index("c")
        row0 = wid * rows_per_w
        flat0 = row0 * seq
        pltpu.sync_copy(x_hbm.at[pl.ds(row0, rows_per_w)], idx_v)

        def gather_copy(i, b):
            r = i * NBUF + b
            return pltpu.make_async_copy(
                table_hbm.at[idx_v.at[r]],
                rows_v.at[b],
                gsem.at[b],
            )

        def store_copy(i, b):
            r = i * NBUF + b
            return pltpu.make_async_copy(
                rows_v.at[b],
                out_hbm.at[pl.ds(flat0 + r * seq, seq), pl.ds(0, DIM)],
                ssem.at[b],
            )

        for b in range(NBUF):
            gather_copy(0, b).start()

        def body(i, carry):
            for b in range(NBUF):
                gather_copy(i, b).wait()
                store_copy(i, b).start()
            for b in range(NBUF):
                store_copy(i, b).wait()
                gather_copy(i + 1, b).start()
            return carry

        lax.fori_loop(0, n_groups - 1, body, 0)

        last = n_groups - 1
        for b in range(NBUF):
            gather_copy(last, b).wait()
            store_copy(last, b).start()
        for b in range(NBUF):
            store_copy(last, b).wait()

    padded = k(x, table)
    return padded[:, :DIM].reshape(batch, seq, DIM)
